# baseline (speedup 1.0000x reference)
import functools

import jax
import jax.numpy as jnp
from jax import lax
from jax.experimental import pallas as pl
from jax.experimental.pallas import tpu as pltpu
from jax.experimental.pallas import tpu_sc as plsc

NC = 2
NS = 16
NW = NC * NS
LANES = 16

CHUNK = 100
NB = 4


def _build(B, T, V, D):
    N = B * T
    assert N % (NW * CHUNK) == 0
    CPW = N // (NW * CHUNK)
    assert (CHUNK * CPW) % T == 0
    assert T == 2 * CHUNK
    assert CPW % NB == 0 and CPW >= 2 * NB and NB % 2 == 0
    JD = D // LANES

    mesh = plsc.VectorSubcoreMesh(
        core_axis_name="c", subcore_axis_name="s", num_cores=NC, num_subcores=NS
    )

    @functools.partial(
        pl.kernel,
        out_type=jax.ShapeDtypeStruct((N // CHUNK, CHUNK, D), jnp.float32),
        mesh=mesh,
        scratch_types=[
            pltpu.VMEM((CPW, CHUNK), jnp.int32),
            pltpu.VMEM((T, D), jnp.float32),
            pltpu.VMEM((NB, CHUNK, D), jnp.float32),
            pltpu.SemaphoreType.DMA,
            pltpu.SemaphoreType.DMA,
            pltpu.SemaphoreType.DMA,
            pltpu.SemaphoreType.DMA,
            pltpu.SemaphoreType.DMA,
            pltpu.SemaphoreType.DMA,
            pltpu.SemaphoreType.DMA,
            pltpu.SemaphoreType.DMA,
        ],
    )
    def emb(x_hbm, tok_hbm, pos_hbm, out_hbm, idx_v, pos_v, buf,
            g0, g1, g2, g3, s0, s1, s2, s3):
        gsems = (g0, g1, g2, g3)
        ssems = (s0, s1, s2, s3)
        wid = lax.axis_index("s") * NC + lax.axis_index("c")
        pltpu.sync_copy(x_hbm.at[wid], idx_v)
        pltpu.sync_copy(pos_hbm, pos_v)
        cbase = wid * CPW

        def issue_gather_add(c, b):
            pltpu.async_copy(
                tok_hbm.at[idx_v.at[c]], buf.at[b], gsems[b], add=True)

        def wait_gather(c, b):
            pltpu.make_async_copy(
                tok_hbm.at[idx_v.at[c]], buf.at[b], gsems[b]).wait()

        def issue_store(c, b):
            pltpu.async_copy(buf.at[b], out_hbm.at[cbase + c], ssems[b])

        def wait_store(c, b):
            pltpu.make_async_copy(
                buf.at[b], out_hbm.at[cbase + c], ssems[b]).wait()

        def refill_pos(b):
            poff = (b % 2) * CHUNK

            @pl.loop(0, CHUNK, unroll=2)
            def _row(r):
                for j in range(JD):
                    sl = pl.ds(j * LANES, LANES)
                    buf[b, r, sl] = pos_v[poff + r, sl]

        for b in range(NB):
            refill_pos(b)
            issue_gather_add(b, b)
            if b > 0:
                wait_gather(b - 1, b - 1)
                issue_store(b - 1, b - 1)

        @pl.loop(NB, CPW, step=NB)
        def _body(c0):
            for b in range(NB):
                c = c0 + b
                bp = (b - 1) % NB
                wait_store(c - NB, b)
                refill_pos(b)
                issue_gather_add(c, b)
                wait_gather(c - 1, bp)
                issue_store(c - 1, bp)

        bl = (CPW - 1) % NB
        wait_gather(CPW - 1, bl)
        issue_store(CPW - 1, bl)

        for b in range(NB):
            wait_store(CPW - NB + b, b)

    return emb


def kernel(x, token_table, pos_table):
    B, T = x.shape
    V, D = token_table.shape
    emb = _build(B, T, V, D)
    x_flat = x.astype(jnp.int32).reshape(NW, (B * T) // (NW * CHUNK), CHUNK)
    out = emb(x_flat, token_table, pos_table)
    return out.reshape(B, T, D)


# speedup vs baseline: 1.1937x; 1.1937x over previous
import functools

import jax
import jax.numpy as jnp
from jax import lax
from jax.experimental import pallas as pl
from jax.experimental.pallas import tpu as pltpu
from jax.experimental.pallas import tpu_sc as plsc

NC = 2
NS = 16
NW = NC * NS
LANES = 16

CHUNK = 100
NB = 2


def _build(B, T, V, D):
    N = B * T
    assert N % (NW * CHUNK) == 0
    CPW = N // (NW * CHUNK)
    assert (CHUNK * CPW) % T == 0
    assert T == NB * CHUNK
    assert CPW % NB == 0 and CPW >= 2 * NB
    JD = D // LANES

    mesh = plsc.VectorSubcoreMesh(
        core_axis_name="c", subcore_axis_name="s", num_cores=NC, num_subcores=NS
    )

    @functools.partial(
        pl.kernel,
        out_type=jax.ShapeDtypeStruct((N // CHUNK, CHUNK, D), jnp.float32),
        mesh=mesh,
        scratch_types=[
            pltpu.VMEM((CPW, CHUNK), jnp.int32),
            pltpu.VMEM((T, D), jnp.float32),
            pltpu.VMEM((NB, CHUNK, D), jnp.float32),
            pltpu.VMEM((NB, CHUNK, D), jnp.float32),
            pltpu.SemaphoreType.DMA,
            pltpu.SemaphoreType.DMA,
            pltpu.SemaphoreType.DMA,
            pltpu.SemaphoreType.DMA,
        ],
    )
    def emb(x_hbm, tok_hbm, pos_hbm, out_hbm, idx_v, pos_v, gbuf, sbuf,
            g0, g1, s0, s1):
        gsems = (g0, g1)
        ssems = (s0, s1)
        wid = lax.axis_index("s") * NC + lax.axis_index("c")
        pltpu.sync_copy(x_hbm.at[wid], idx_v)
        pltpu.sync_copy(pos_hbm, pos_v)
        cbase = wid * CPW

        def issue_gather(c, b):
            pltpu.async_copy(tok_hbm.at[idx_v.at[c]], gbuf.at[b], gsems[b])

        def wait_gather(c, b):
            pltpu.make_async_copy(
                tok_hbm.at[idx_v.at[c]], gbuf.at[b], gsems[b]).wait()

        def issue_store(c, b):
            pltpu.async_copy(sbuf.at[b], out_hbm.at[cbase + c], ssems[b])

        def wait_store(c, b):
            pltpu.make_async_copy(
                sbuf.at[b], out_hbm.at[cbase + c], ssems[b]).wait()

        def add_pos(b):
            poff = b * CHUNK

            @pl.loop(0, CHUNK)
            def _row(r):
                for j in range(JD):
                    sl = pl.ds(j * LANES, LANES)
                    sbuf[b, r, sl] = gbuf[b, r, sl] + pos_v[poff + r, sl]

        for b in range(NB):
            issue_gather(b, b)

        for b in range(NB):
            wait_gather(b, b)
            add_pos(b)
            issue_gather(b + NB, b)
            issue_store(b, b)

        @pl.loop(NB, CPW - NB, step=NB)
        def _body(c0):
            for b in range(NB):
                c = c0 + b
                wait_gather(c, b)
                wait_store(c - NB, b)
                add_pos(b)
                issue_gather(c + NB, b)
                issue_store(c, b)

        for b in range(NB):
            c = CPW - NB + b
            wait_gather(c, b)
            wait_store(c - NB, b)
            add_pos(b)
            issue_store(c, b)

        for b in range(NB):
            wait_store(CPW - NB + b, b)

    return emb


def kernel(x, token_table, pos_table):
    B, T = x.shape
    V, D = token_table.shape
    emb = _build(B, T, V, D)
    x_flat = x.astype(jnp.int32).reshape(NW, (B * T) // (NW * CHUNK), CHUNK)
    out = emb(x_flat, token_table, pos_table)
    return out.reshape(B, T, D)
